# baseline (speedup 1.0000x reference)
import functools

import jax
import jax.numpy as jnp
from jax import lax
from jax.experimental import pallas as pl
from jax.experimental.pallas import tpu as pltpu
from jax.experimental.pallas import tpu_sc as plsc

NC = 2
NS = 16
L = 16
NW = NC * NS

CH = 128


def _make_sc_kernel(B, K, HM):
    assert B % NW == 0
    bw = B // NW
    nch = bw // CH
    gpc = CH // L
    ng = bw // L
    assert nch * CH == bw and K % L == 0
    K2 = 2 * K

    mesh = plsc.VectorSubcoreMesh(core_axis_name="c", subcore_axis_name="s")

    @functools.partial(
        pl.kernel,
        mesh=mesh,
        out_type=jax.ShapeDtypeStruct((B,), jnp.float32),
        compiler_params=pltpu.CompilerParams(
            needs_layout_passes=False, use_tc_tiling_on_sc=True),
        scratch_types=[
            pltpu.VMEM((bw,), jnp.int32),
            pltpu.VMEM((bw,), jnp.int32),
            pltpu.VMEM((bw,), jnp.int32),
            pltpu.VMEM((2, L, K), jnp.float32),
            pltpu.VMEM((CH, K2), jnp.float32),
            pltpu.VMEM((bw,), jnp.float32),
            pltpu.SemaphoreType.DMA((2,)),
            pltpu.SemaphoreType.DMA,
        ],
    )
    def body(users_hbm, movies_hbm, uemb_hbm, mpairs_hbm, out_hbm,
             uidx, mg, mb, urows, mbuf, outv, usem, msem):
        wid = lax.axis_index("s") * NC + lax.axis_index("c")
        base = wid * bw
        pltpu.sync_copy(users_hbm.at[pl.ds(base, bw)], uidx)
        pltpu.sync_copy(movies_hbm.at[pl.ds(base, bw)], mg)
        lane = lax.iota(jnp.int32, L)

        def split(j, _):
            sl = pl.ds(j * L, L)
            mv = mg[sl]
            mb[sl] = jnp.where(mv < HM, 0, K)
            mg[sl] = jnp.where(mv < HM, mv, mv - HM)
            return _

        lax.fori_loop(0, bw // L, split, 0)

        def fetch(g, s):
            uvec = uidx[pl.ds(g * L, L)]
            for i in range(L):
                pltpu.async_copy(
                    uemb_hbm.at[uvec[i]], urows.at[s, i], usem.at[s])

        def drain(s):
            for _ in range(L):
                pltpu.make_async_copy(
                    uemb_hbm.at[0], urows.at[0, 0], usem.at[s]).wait()

        fetch(0, 0)

        for c in range(nch):
            pltpu.async_copy(
                mpairs_hbm.at[mg.at[pl.ds(c * CH, CH)]], mbuf, msem).wait()

            def group(gl, carry, c=c):
                g = c * gpc + gl
                s = g % 2

                @pl.when(g + 1 < ng)
                def _():
                    fetch(g + 1, (g + 1) % 2)

                drain(s)
                mbv = mb[pl.ds(g * L, L)]
                accv = jnp.zeros((L,), jnp.float32)
                for i in range(L):
                    r = gl * L + i
                    mbase = mbv[i]
                    p = (urows[s, i, pl.ds(0, L)]
                         * mbuf[r, pl.ds(mbase, L)])
                    for k in range(L, K, L):
                        p += (urows[s, i, pl.ds(k, L)]
                              * mbuf[r, pl.ds(mbase + k, L)])
                    accv = jnp.where(lane == i, plsc.cumsum(p)[L - 1], accv)
                outv[pl.ds(g * L, L)] = accv
                return carry

            lax.fori_loop(0, gpc, group, 0)

        pltpu.sync_copy(outv, out_hbm.at[pl.ds(base, bw)])

    return body


def kernel(users, movies, user_emb, movie_emb):
    B = users.shape[0]
    K = user_emb.shape[1]
    hm = movie_emb.shape[0] // 2
    mpairs = jnp.concatenate([movie_emb[:hm], movie_emb[hm:]], axis=1)
    return _make_sc_kernel(B, K, hm)(
        users.astype(jnp.int32), movies.astype(jnp.int32),
        user_emb, mpairs)
